# Initial kernel scaffold; baseline (speedup 1.0000x reference)
import jax
import jax.numpy as jnp
from jax.experimental import pallas as pl


def kernel(ze, codebook):
    raise NotImplementedError("write your pallas kernel here")


# speedup vs baseline: 1.1154x; 1.1154x over previous
import functools

import jax
import jax.numpy as jnp
from jax import lax
from jax.experimental import pallas as pl
from jax.experimental.pallas import tpu as pltpu
from jax.experimental.pallas import tpu_sc as plsc

_N_EMB = 1024
_D = 64
_TOK = 32 * 576
_BLK = 512
_NBLK = _TOK // _BLK

_NW = 32
_BPW = _TOK // _NW
_CH = 96
_NCH = _BPW // _CH


def _argmin_body(ze_ref, cbt_ref, am_ref, ent_ref, loss_ref, counts_ref, acc_ref):
    i = pl.program_id(0)
    ze = ze_ref[...]
    cbt = cbt_ref[...]
    a = jnp.sum(ze * ze, axis=1, keepdims=True)
    b = jnp.sum(cbt * cbt, axis=0, keepdims=True)
    mm = lax.dot_general(ze, cbt, (((1,), (0,)), ((), ())),
                         preferred_element_type=jnp.float32)
    sq = (a + b) - 2.0 * mm
    m = jnp.min(sq, axis=1)
    iota = lax.broadcasted_iota(jnp.int32, (_BLK, _N_EMB), 1)
    am = jnp.min(jnp.where(sq == m[:, None], iota, _N_EMB), axis=1)
    am_ref[...] = am.reshape(1, 1, _BLK)
    onehot = (iota == am[:, None]).astype(jnp.float32)

    @pl.when(i == 0)
    def _init():
        counts_ref[...] = jnp.zeros_like(counts_ref)
        acc_ref[0] = 0.0

    counts_ref[...] += jnp.sum(onehot, axis=0, keepdims=True)
    acc_ref[0] += jnp.sum(m)

    @pl.when(i == _NBLK - 1)
    def _finish():
        probs = counts_ref[0, :] / 10.0
        ent_ref[...] = jnp.sum(probs * jnp.log(probs + 1e-10)).reshape(1, 1)
        loss_ref[...] = (acc_ref[0] / float(_TOK * _D)).reshape(1, 1)


def _argmin_losses(ze2d, cbt):
    return pl.pallas_call(
        _argmin_body,
        grid=(_NBLK,),
        in_specs=[
            pl.BlockSpec((_BLK, _D), lambda i: (i, 0)),
            pl.BlockSpec((_D, _N_EMB), lambda i: (0, 0)),
        ],
        out_specs=[
            pl.BlockSpec((1, 1, _BLK), lambda i: (i, 0, 0)),
            pl.BlockSpec((1, 1), lambda i: (0, 0)),
            pl.BlockSpec((1, 1), lambda i: (0, 0)),
        ],
        out_shape=[
            jax.ShapeDtypeStruct((_NBLK, 1, _BLK), jnp.int32),
            jax.ShapeDtypeStruct((1, 1), jnp.float32),
            jax.ShapeDtypeStruct((1, 1), jnp.float32),
        ],
        scratch_shapes=[
            pltpu.VMEM((1, _N_EMB), jnp.float32),
            pltpu.SMEM((1,), jnp.float32),
        ],
    )(ze2d, cbt)


def _sc_gather(codebook, idx2d):
    mesh = plsc.VectorSubcoreMesh(core_axis_name="c", subcore_axis_name="s")

    @functools.partial(
        pl.kernel,
        out_type=jax.ShapeDtypeStruct((_TOK, 2 * _D), jnp.float32),
        mesh=mesh,
        scratch_types=[
            pltpu.VMEM((_NCH, _CH), jnp.int32),
            pltpu.VMEM((_BPW, 2 * _D), jnp.float32),
            pltpu.SemaphoreType.DMA,
        ],
    )
    def gather_kernel(cb_hbm, idx_hbm, out_hbm, idx_v, rows_v, sem):
        wid = lax.axis_index("s") * 2 + lax.axis_index("c")
        pltpu.sync_copy(idx_hbm.at[wid], idx_v)
        copies = [
            pltpu.async_copy(cb_hbm.at[idx_v.at[j]],
                             rows_v.at[pl.ds(j * _CH, _CH)], sem)
            for j in range(_NCH)
        ]
        for c in copies:
            c.wait()
        pltpu.sync_copy(rows_v, out_hbm.at[pl.ds(wid * _BPW, _BPW)])

    cb_pad = jnp.pad(codebook, ((0, 0), (0, _D)))
    return gather_kernel(cb_pad, idx2d)


def kernel(ze, codebook):
    ze2d = ze.reshape(_TOK, _D)
    cbt = codebook.T
    am3d, ent, loss = _argmin_losses(ze2d, cbt)
    am_flat = am3d.reshape(_TOK)
    zq = _sc_gather(codebook, am_flat.reshape(_NW, _NCH, _CH))[:, :_D]
    argmin = am_flat.reshape(ze.shape[0], ze.shape[1])
    vq_e_loss = loss[0, 0]
    return (argmin, zq.reshape(ze.shape), vq_e_loss, vq_e_loss, ent[0, 0])
